# baseline (speedup 1.0000x reference)
import jax
import jax.numpy as jnp
from jax.experimental import pallas as pl
from jax.experimental.pallas import tpu as pltpu


def _tg_kernel(
    z_ref,
    g_wih, g_whh, g_bih, g_bhh, g_wl, g_bl,
    s_wih, s_whh, s_bih, s_bhh, s_wl, s_bl,
    d_wih, d_whh, d_bih, d_bhh, wstack,
    out_ref,
    gi_ref, hcat_ref,
):
    Bb, T, Dz = z_ref.shape
    H = g_whh.shape[0]
    G3 = 3 * H
    TB = T * Bb
    f32 = jnp.float32
    bf16 = jnp.bfloat16

    def fold_rz(bih_ref, bhh_ref):
        bih = bih_ref[...]
        bhh = bhh_ref[...]
        b = bih + jnp.concatenate(
            [bhh[:, :2 * H], jnp.zeros((1, H), f32)], axis=1)
        return b, bhh[:, 2 * H:]

    g_b, g_bhn = fold_rz(g_bih, g_bhh)
    s_b, s_bhn = fold_rz(s_bih, s_bhh)
    d_b, d_bhn = fold_rz(d_bih, d_bhh)

    whh1, whh2, whh3 = g_whh[...], s_whh[...], d_whh[...]
    wl1 = g_wl[...].astype(bf16)
    wl2 = s_wl[...].astype(bf16)
    wih2 = s_wih[...].astype(bf16)
    wih3 = d_wih[...].astype(bf16)
    bl1, bl2 = g_bl[...], s_bl[...]

    def gru_step(gi, h, whh, bhn):
        gh = jnp.dot(h, whh, preferred_element_type=f32)
        r = jax.nn.sigmoid(gi[:, :H] + gh[:, :H])
        zg = jax.nn.sigmoid(gi[:, H:2 * H] + gh[:, H:2 * H])
        n = jnp.tanh(gi[:, 2 * H:] + r * (gh[:, 2 * H:] + bhn))
        return n + zg * (h - n)

    def couple(h, wl, bl, wih, b):
        x = jax.nn.sigmoid(
            jnp.dot(h.astype(bf16), wl, preferred_element_type=f32) + bl)
        return jnp.dot(x.astype(bf16), wih, preferred_element_type=f32) + b

    z_flat = jnp.swapaxes(z_ref[...], 0, 1).reshape(TB, Dz).astype(bf16)
    gi_ref[...] = (
        jnp.dot(z_flat, g_wih[...].astype(bf16), preferred_element_type=f32)
        + g_b
    ).reshape(T, Bb, G3)

    h1 = jnp.zeros((Bb, H), f32)
    h2 = jnp.zeros((Bb, H), f32)
    h3 = jnp.zeros((Bb, H), f32)
    for t in range(T):
        h1 = gru_step(gi_ref[t], h1, whh1, g_bhn)
        gi2 = couple(h1, wl1, bl1, wih2, s_b)
        h2 = gru_step(gi2, h2, whh2, s_bhn)
        gi3 = couple(h2, wl2, bl2, wih3, d_b)
        h3 = gru_step(gi3, h3, whh3, d_bhn)
        hcat_ref[:, t * H:(t + 1) * H] = h3

    out_ref[...] = jnp.dot(
        hcat_ref[...], wstack[...], preferred_element_type=f32)


def _block_b(B):
    for c in (512, 256, 128, 64, 32, 16, 8):
        if B % c == 0:
            return c
    return B


def kernel(Z, g_wih, g_whh, g_bih, g_bhh, g_wl, g_bl,
           s_wih, s_whh, s_bih, s_bhh, s_wl, s_bl,
           d_wih, d_whh, d_bih, d_bhh, d_wl, d_bl):
    B, T, Dz = Z.shape
    H = g_whh.shape[0]
    f32 = jnp.float32

    eye = jnp.eye(T, dtype=f32)
    wstack = (eye[:, None, :] * d_wl[None, :, 0, None]).reshape(T * H, T)
    wstack = jnp.pad(wstack, ((0, 0), (0, 128 - T)))

    weights = [g_wih, g_whh, g_bih, g_bhh, g_wl, g_bl,
               s_wih, s_whh, s_bih, s_bhh, s_wl, s_bl,
               d_wih, d_whh, d_bih, d_bhh, wstack]
    weights = [w.astype(f32) for w in weights]

    def fwd(Zl, *wl):
        Bl = Zl.shape[0]
        Bb = _block_b(Bl)
        ws, bias0 = wl[:-1], wl[-1]

        def wspec(w):
            return pl.BlockSpec(w.shape, lambda i, nd=w.ndim: (0,) * nd)

        out = pl.pallas_call(
            _tg_kernel,
            out_shape=jax.ShapeDtypeStruct((Bl, 128), f32),
            grid_spec=pltpu.PrefetchScalarGridSpec(
                num_scalar_prefetch=0,
                grid=(Bl // Bb,),
                in_specs=[pl.BlockSpec((Bb, T, Dz), lambda i: (i, 0, 0))]
                + [wspec(w) for w in ws],
                out_specs=pl.BlockSpec((Bb, 128), lambda i: (i, 0)),
                scratch_shapes=[
                    pltpu.VMEM((T, Bb, 3 * H), f32),
                    pltpu.VMEM((Bb, T * H), f32),
                ],
            ),
            compiler_params=pltpu.CompilerParams(
                dimension_semantics=("parallel",)),
        )(Zl, *ws)

        y = out[:, :T] + bias0[0, 0]
        return y[:, :, None]

    return fwd(Z, *weights, d_bl.astype(f32))


# speedup vs baseline: 1.2021x; 1.2021x over previous
import jax
import jax.numpy as jnp
import numpy as np
from jax.experimental import pallas as pl
from jax.experimental.pallas import tpu as pltpu
from jax.sharding import Mesh, PartitionSpec as P


def _tg_kernel(
    z_ref,
    g_wih, g_whh, g_bih, g_bhh, g_wl, g_bl,
    s_wih, s_whh, s_bih, s_bhh, s_wl, s_bl,
    d_wih, d_whh, d_bih, d_bhh, wstack,
    out_ref,
    gi_ref, h_ref, hcat_ref,
):
    Bb, T, Dz = z_ref.shape
    H = g_whh.shape[0]
    G3 = 3 * H
    TB = T * Bb
    f32 = jnp.float32
    bf16 = jnp.bfloat16

    def fold_rz(bih_ref, bhh_ref):
        bih = bih_ref[...]
        bhh = bhh_ref[...]
        b = bih + jnp.concatenate(
            [bhh[:, :2 * H], jnp.zeros((1, H), f32)], axis=1)
        return b, bhh[:, 2 * H:]

    def run_gru(whh_ref, bhn, store_h):
        whh = whh_ref[...]
        Bh = Bb // 4
        hs = [jnp.zeros((Bh, H), f32) for _ in range(4)]
        for t in range(T):
            for i in range(4):
                h = hs[i]
                gi = gi_ref[t, i * Bh:(i + 1) * Bh, :]
                gh = jnp.dot(h, whh, preferred_element_type=f32)
                r = jax.nn.sigmoid(gi[:, :H] + gh[:, :H])
                zg = jax.nn.sigmoid(gi[:, H:2 * H] + gh[:, H:2 * H])
                n = jnp.tanh(gi[:, 2 * H:] + r * (gh[:, 2 * H:] + bhn))
                hs[i] = n + zg * (h - n)
                store_h(t, i * Bh, hs[i])

    def store_tm(t, b0, h):
        h_ref[t, b0:b0 + h.shape[0], :] = h

    def store_cat(t, b0, h):
        hcat_ref[b0:b0 + h.shape[0], t * H:(t + 1) * H] = h

    g_b, g_bhn = fold_rz(g_bih, g_bhh)
    s_b, s_bhn = fold_rz(s_bih, s_bhh)
    d_b, d_bhn = fold_rz(d_bih, d_bhh)

    z_flat = jnp.swapaxes(z_ref[...], 0, 1).reshape(TB, Dz).astype(bf16)
    gi_ref[...] = (
        jnp.dot(z_flat, g_wih[...].astype(bf16), preferred_element_type=f32)
        + g_b
    ).reshape(T, Bb, G3)

    run_gru(g_whh, g_bhn, store_tm)

    h1 = h_ref[...].reshape(TB, H).astype(bf16)
    e_hat = jax.nn.sigmoid(
        jnp.dot(h1, g_wl[...].astype(bf16), preferred_element_type=f32)
        + g_bl[...]).astype(bf16)
    gi_ref[...] = (
        jnp.dot(e_hat, s_wih[...].astype(bf16), preferred_element_type=f32)
        + s_b
    ).reshape(T, Bb, G3)

    run_gru(s_whh, s_bhn, store_tm)

    h2 = h_ref[...].reshape(TB, H).astype(bf16)
    h_hat = jax.nn.sigmoid(
        jnp.dot(h2, s_wl[...].astype(bf16), preferred_element_type=f32)
        + s_bl[...]).astype(bf16)
    gi_ref[...] = (
        jnp.dot(h_hat, d_wih[...].astype(bf16), preferred_element_type=f32)
        + d_b
    ).reshape(T, Bb, G3)

    run_gru(d_whh, d_bhn, store_cat)

    out_ref[...] = jnp.dot(
        hcat_ref[...], wstack[...], preferred_element_type=f32)


def _block_b(B):
    for c in (512, 256, 128, 64, 32, 16, 8):
        if B % c == 0:
            return c
    return B


def kernel(Z, g_wih, g_whh, g_bih, g_bhh, g_wl, g_bl,
           s_wih, s_whh, s_bih, s_bhh, s_wl, s_bl,
           d_wih, d_whh, d_bih, d_bhh, d_wl, d_bl):
    B, T, Dz = Z.shape
    H = g_whh.shape[0]
    f32 = jnp.float32

    eye = jnp.eye(T, dtype=f32)
    wstack = (eye[:, None, :] * d_wl[None, :, 0, None]).reshape(T * H, T)
    wstack = jnp.pad(wstack, ((0, 0), (0, 128 - T)))

    weights = [g_wih, g_whh, g_bih, g_bhh, g_wl, g_bl,
               s_wih, s_whh, s_bih, s_bhh, s_wl, s_bl,
               d_wih, d_whh, d_bih, d_bhh, wstack]
    weights = [w.astype(f32) for w in weights]

    def fwd(Zl, *wl):
        Bl = Zl.shape[0]
        Bb = _block_b(Bl)
        ws, bias0 = wl[:-1], wl[-1]

        def wspec(w):
            return pl.BlockSpec(w.shape, lambda i, nd=w.ndim: (0,) * nd)

        out = pl.pallas_call(
            _tg_kernel,
            out_shape=jax.ShapeDtypeStruct((Bl, 128), f32),
            grid_spec=pltpu.PrefetchScalarGridSpec(
                num_scalar_prefetch=0,
                grid=(Bl // Bb,),
                in_specs=[pl.BlockSpec((Bb, T, Dz), lambda i: (i, 0, 0))]
                + [wspec(w) for w in ws],
                out_specs=pl.BlockSpec((Bb, 128), lambda i: (i, 0)),
                scratch_shapes=[
                    pltpu.VMEM((T, Bb, 3 * H), f32),
                    pltpu.VMEM((T, Bb, H), f32),
                    pltpu.VMEM((Bb, T * H), f32),
                ],
            ),
            compiler_params=pltpu.CompilerParams(
                dimension_semantics=("parallel",)),
        )(Zl, *ws)

        y = out[:, :T] + bias0[0, 0]
        return y[:, :, None]

    return fwd(Z, *weights, d_bl.astype(f32))


# speedup vs baseline: 1.2278x; 1.0214x over previous
import jax
import jax.numpy as jnp
from jax.experimental import pallas as pl
from jax.experimental.pallas import tpu as pltpu


def _tg_kernel(
    z_ref,
    g_wih, g_whh, g_bih, g_bhh, g_wl, g_bl,
    s_wih, s_whh, s_bih, s_bhh, s_wl, s_bl,
    d_wih, d_whh, d_bih, d_bhh, wstack,
    out_ref,
    gi_ref, h_ref, hcat_ref,
):
    Bb, T, Dz = z_ref.shape
    H = g_whh.shape[0]
    G3 = 3 * H
    TB = T * Bb
    f32 = jnp.float32
    bf16 = jnp.bfloat16

    def fold_rz(bih_ref, bhh_ref):
        bih = bih_ref[...]
        bhh = bhh_ref[...]
        b = bih + jnp.concatenate(
            [bhh[:, :2 * H], jnp.zeros((1, H), f32)], axis=1)
        return b, bhh[:, 2 * H:]

    def run_gru(whh_ref, bhn, store_h):
        whh = whh_ref[...]
        Bh = Bb // 2
        hs = [jnp.zeros((Bh, H), f32), jnp.zeros((Bh, H), f32)]
        for t in range(T):
            for i in range(2):
                h = hs[i]
                gi = gi_ref[t, i * Bh:(i + 1) * Bh, :]
                gh = jnp.dot(h, whh, preferred_element_type=f32)
                r = jax.nn.sigmoid(gi[:, :H] + gh[:, :H])
                zg = jax.nn.sigmoid(gi[:, H:2 * H] + gh[:, H:2 * H])
                n = jnp.tanh(gi[:, 2 * H:] + r * (gh[:, 2 * H:] + bhn))
                hs[i] = n + zg * (h - n)
                store_h(t, i * Bh, hs[i])

    def store_tm(t, b0, h):
        h_ref[t, b0:b0 + h.shape[0], :] = h

    def store_cat(t, b0, h):
        hcat_ref[b0:b0 + h.shape[0], t * H:(t + 1) * H] = h

    g_b, g_bhn = fold_rz(g_bih, g_bhh)
    s_b, s_bhn = fold_rz(s_bih, s_bhh)
    d_b, d_bhn = fold_rz(d_bih, d_bhh)

    z_flat = jnp.swapaxes(z_ref[...], 0, 1).reshape(TB, Dz).astype(bf16)
    gi_ref[...] = (
        jnp.dot(z_flat, g_wih[...].astype(bf16), preferred_element_type=f32)
        + g_b
    ).reshape(T, Bb, G3)

    run_gru(g_whh, g_bhn, store_tm)

    h1 = h_ref[...].reshape(TB, H).astype(bf16)
    e_hat = jax.nn.sigmoid(
        jnp.dot(h1, g_wl[...].astype(bf16), preferred_element_type=f32)
        + g_bl[...]).astype(bf16)
    gi_ref[...] = (
        jnp.dot(e_hat, s_wih[...].astype(bf16), preferred_element_type=f32)
        + s_b
    ).reshape(T, Bb, G3)

    run_gru(s_whh, s_bhn, store_tm)

    h2 = h_ref[...].reshape(TB, H).astype(bf16)
    h_hat = jax.nn.sigmoid(
        jnp.dot(h2, s_wl[...].astype(bf16), preferred_element_type=f32)
        + s_bl[...]).astype(bf16)
    gi_ref[...] = (
        jnp.dot(h_hat, d_wih[...].astype(bf16), preferred_element_type=f32)
        + d_b
    ).reshape(T, Bb, G3)

    run_gru(d_whh, d_bhn, store_cat)

    out_ref[...] = jnp.dot(
        hcat_ref[...], wstack[...], preferred_element_type=f32)


def _block_b(B):
    for c in (512, 256, 128, 64, 32, 16, 8):
        if B % c == 0:
            return c
    return B


def kernel(Z, g_wih, g_whh, g_bih, g_bhh, g_wl, g_bl,
           s_wih, s_whh, s_bih, s_bhh, s_wl, s_bl,
           d_wih, d_whh, d_bih, d_bhh, d_wl, d_bl):
    B, T, Dz = Z.shape
    H = g_whh.shape[0]
    f32 = jnp.float32

    eye = jnp.eye(T, dtype=f32)
    wstack = (eye[:, None, :] * d_wl[None, :, 0, None]).reshape(T * H, T)
    wstack = jnp.pad(wstack, ((0, 0), (0, 128 - T)))

    weights = [g_wih, g_whh, g_bih, g_bhh, g_wl, g_bl,
               s_wih, s_whh, s_bih, s_bhh, s_wl, s_bl,
               d_wih, d_whh, d_bih, d_bhh, wstack]
    weights = [w.astype(f32) for w in weights]

    def fwd(Zl, *wl):
        Bl = Zl.shape[0]
        Bb = _block_b(Bl)
        ws, bias0 = wl[:-1], wl[-1]

        def wspec(w):
            return pl.BlockSpec(w.shape, lambda i, nd=w.ndim: (0,) * nd)

        out = pl.pallas_call(
            _tg_kernel,
            out_shape=jax.ShapeDtypeStruct((Bl, 128), f32),
            grid_spec=pltpu.PrefetchScalarGridSpec(
                num_scalar_prefetch=0,
                grid=(Bl // Bb,),
                in_specs=[pl.BlockSpec((Bb, T, Dz), lambda i: (i, 0, 0))]
                + [wspec(w) for w in ws],
                out_specs=pl.BlockSpec((Bb, 128), lambda i: (i, 0)),
                scratch_shapes=[
                    pltpu.VMEM((T, Bb, 3 * H), f32),
                    pltpu.VMEM((T, Bb, H), f32),
                    pltpu.VMEM((Bb, T * H), f32),
                ],
            ),
            compiler_params=pltpu.CompilerParams(
                dimension_semantics=("parallel",)),
        )(Zl, *ws)

        y = out[:, :T] + bias0[0, 0]
        return y[:, :, None]

    return fwd(Z, *weights, d_bl.astype(f32))
